# Initial kernel scaffold; baseline (speedup 1.0000x reference)
import jax
import jax.numpy as jnp
from jax.experimental import pallas as pl


def kernel(x, edge_index, W0, al0, ar0, b0, W1, al1, ar1, b1, W2, al2, ar2, b2, P1W, P1b, P2W):
    raise NotImplementedError("write your pallas kernel here")


# speedup vs baseline: 23.0070x; 23.0070x over previous
import functools

import jax
import jax.numpy as jnp
from jax import lax
from jax.experimental import pallas as pl
from jax.experimental.pallas import tpu as pltpu
from jax.experimental.pallas import tpu_sc as plsc

N = 10000
E = 320000
IN_FEATS = 128
D = 64
DT = 128
HID = 16

NC = 2
NS = 16
NW = NC * NS
K = 128
CHUNKS = (E + NW * K - 1) // (NW * K)
EP = NW * K * CHUNKS
EPT = K * CHUNKS
NPADROWS = 112
NPAD = N + NPADROWS
RPT = NPAD // NS
ZB = 80
ZCHUNKS = [(o, 80) for o in range(0, 560, 80)] + [(560, 72)]

R = 1000


def _sc_edge_body(feat_hbm, erp_hbm, srcp_hbm, dstp_hbm,
                  num_out,
                  er_v, sidx_v, didx_v, rows_v, ex_v,
                  zrow_v, shared_num, sem):
    cid = lax.axis_index("c")
    sid = lax.axis_index("s")
    gwid = cid * NS + sid

    pltpu.sync_copy(erp_hbm, er_v)

    zero16 = jnp.zeros((16,), jnp.float32)

    def zero_body(r, _):
        for j in range(DT // 16):
            zrow_v[r, pl.ds(j * 16, 16)] = zero16
        return 0

    lax.fori_loop(0, ZB, zero_body, 0)

    for (off, sz) in ZCHUNKS:
        pltpu.sync_copy(zrow_v.at[0:sz], shared_num.at[pl.ds(sid * RPT + off, sz)])
    plsc.subcore_barrier()

    lane = lax.iota(jnp.int32, 16)
    col0 = jnp.zeros((16,), jnp.int32)

    def chunk_body(c, _):
        base = gwid * EPT + c * K
        pltpu.sync_copy(srcp_hbm.at[pl.ds(base, K)], sidx_v)
        pltpu.sync_copy(dstp_hbm.at[pl.ds(base, K)], didx_v)
        pltpu.async_copy(feat_hbm.at[sidx_v], rows_v, sem).wait()

        for g in range(K // 16):
            didx = didx_v[pl.ds(g * 16, 16)]
            elv = plsc.load_gather(rows_v, [g * 16 + lane, col0 + (D + 1)])
            e = elv + plsc.load_gather(er_v, [didx])
            e = jnp.where(e > 0.0, e, 0.2 * e)
            ex_v[pl.ds(g * 16, 16)] = jnp.exp(e)

        def mul_body(k, _):
            bro = plsc.load_gather(ex_v, [col0 + k])
            for j in range(5):
                rows_v[k, pl.ds(j * 16, 16)] = rows_v[k, pl.ds(j * 16, 16)] * bro
            return 0

        lax.fori_loop(0, K, mul_body, 0)

        pltpu.sync_copy(rows_v, shared_num.at[didx_v], add=True)
        return 0

    lax.fori_loop(0, CHUNKS, chunk_body, 0)
    plsc.subcore_barrier()

    for (off, sz) in ZCHUNKS:
        pltpu.sync_copy(shared_num.at[pl.ds(sid * RPT + off, sz)], zrow_v.at[0:sz])
        pltpu.sync_copy(zrow_v.at[0:sz], num_out.at[cid, pl.ds(sid * RPT + off, sz)])


_sc_edge = functools.partial(
    pl.kernel,
    out_type=jax.ShapeDtypeStruct((NC, NPAD, DT), jnp.float32),
    mesh=plsc.VectorSubcoreMesh(core_axis_name="c", subcore_axis_name="s"),
    compiler_params=pltpu.CompilerParams(needs_layout_passes=False),
    scratch_types=[
        pltpu.VMEM((NPAD,), jnp.float32),
        pltpu.VMEM((K,), jnp.int32),
        pltpu.VMEM((K,), jnp.int32),
        pltpu.VMEM((K, DT), jnp.float32),
        pltpu.VMEM((K,), jnp.float32),
        pltpu.VMEM((ZB, DT), jnp.float32),
        pltpu.VMEM_SHARED((NPAD, DT), jnp.float32),
        pltpu.SemaphoreType.DMA,
    ],
)(_sc_edge_body)


def _l2norm(h):
    n = jnp.sqrt(jnp.sum(h * h, axis=1, keepdims=True))
    return h / jnp.maximum(n, 1e-12)


def _emit(feat, al_ref, ar_ref, feat_ref, er_ref):
    r = feat.shape[0]
    el = jnp.sum(feat * al_ref[...], axis=1, keepdims=True)
    feat_ref[...] = jnp.concatenate(
        [feat, jnp.ones((r, 1), jnp.float32), el,
         jnp.zeros((r, DT - D - 2), jnp.float32)], axis=1)
    er_ref[...] = jnp.sum(feat * ar_ref[...], axis=1, keepdims=True)


def _pre0_body(x_ref, w_ref, al_ref, ar_ref, feat_ref, er_ref):
    hn = _l2norm(x_ref[...])
    feat = jnp.dot(hn, w_ref[...], preferred_element_type=jnp.float32)
    _emit(feat, al_ref, ar_ref, feat_ref, er_ref)


def _tc_pre0(x, w, al, ar):
    return pl.pallas_call(
        _pre0_body,
        grid=(N // R,),
        in_specs=[
            pl.BlockSpec((R, IN_FEATS), lambda i: (i, 0)),
            pl.BlockSpec((IN_FEATS, D), lambda i: (0, 0)),
            pl.BlockSpec((1, D), lambda i: (0, 0)),
            pl.BlockSpec((1, D), lambda i: (0, 0)),
        ],
        out_specs=[
            pl.BlockSpec((R, DT), lambda i: (i, 0)),
            pl.BlockSpec((R, 1), lambda i: (i, 0)),
        ],
        out_shape=[
            jax.ShapeDtypeStruct((N, DT), jnp.float32),
            jax.ShapeDtypeStruct((N, 1), jnp.float32),
        ],
    )(x, w, al, ar)


def _combine(n0_ref, n1_ref, b_ref):
    n0 = n0_ref[...]
    n1 = n1_ref[...]
    s = jnp.maximum(n0[:, D:D + 1] + n1[:, D:D + 1], 1e-12)
    return (n0[:, 0:D] + n1[:, 0:D]) / s + b_ref[...]


def _make_pre_mid(with_prev):
    def body(*refs):
        if with_prev:
            (n0_ref, n1_ref, hprev_ref, b_ref,
             w_ref, al_ref, ar_ref, feat_ref, er_ref, hacc_ref) = refs
        else:
            (n0_ref, n1_ref, b_ref,
             w_ref, al_ref, ar_ref, feat_ref, er_ref, hacc_ref) = refs
        rst = _combine(n0_ref, n1_ref, b_ref)
        hacc = rst + hprev_ref[...] if with_prev else rst
        hacc_ref[...] = hacc
        hn = _l2norm(hacc)
        feat = jnp.dot(hn, w_ref[...], preferred_element_type=jnp.float32)
        _emit(feat, al_ref, ar_ref, feat_ref, er_ref)

    rt = pl.BlockSpec((R, DT), lambda i: (i, 0))
    rd = pl.BlockSpec((R, D), lambda i: (i, 0))
    r1 = pl.BlockSpec((R, 1), lambda i: (i, 0))
    full1d = pl.BlockSpec((1, D), lambda i: (0, 0))
    in_specs = [rt, rt] + ([rd] if with_prev else []) + [
        full1d, pl.BlockSpec((D, D), lambda i: (0, 0)), full1d, full1d]

    def run(*args):
        return pl.pallas_call(
            body,
            grid=(N // R,),
            in_specs=in_specs,
            out_specs=[rt, r1, rd],
            out_shape=[
                jax.ShapeDtypeStruct((N, DT), jnp.float32),
                jax.ShapeDtypeStruct((N, 1), jnp.float32),
                jax.ShapeDtypeStruct((N, D), jnp.float32),
            ],
        )(*args)

    return run


_tc_pre1 = _make_pre_mid(False)
_tc_pre2 = _make_pre_mid(True)


def _score(h, p1w_ref, p1b_ref, p2w_ref):
    t = jnp.tanh(jnp.dot(h, p1w_ref[...], preferred_element_type=jnp.float32)
                 + p1b_ref[...])
    return jnp.sum(t * p2w_ref[...])


def _final_a_body(n0_ref, n1_ref, hB_ref, hA_ref, b_ref,
                  p1w_ref, p1b_ref, p2w_ref, hC_ref, wsum_ref):
    i = pl.program_id(0)
    rst = _combine(n0_ref, n1_ref, b_ref)
    hC = hB_ref[...] + rst
    hC_ref[...] = hC
    w0 = _score(hA_ref[...], p1w_ref, p1b_ref, p2w_ref)
    w1 = _score(hB_ref[...], p1w_ref, p1b_ref, p2w_ref)
    w2 = _score(hC, p1w_ref, p1b_ref, p2w_ref)
    part = jnp.broadcast_to(jnp.stack([w0, w1, w2])[:, None], (3, 128))

    @pl.when(i == 0)
    def _():
        wsum_ref[...] = jnp.zeros_like(wsum_ref)

    wsum_ref[...] += part


def _tc_final_a(n0, n1, hB, hA, b, p1w, p1b, p2w):
    rt = pl.BlockSpec((R, DT), lambda i: (i, 0))
    rd = pl.BlockSpec((R, D), lambda i: (i, 0))
    return pl.pallas_call(
        _final_a_body,
        grid=(N // R,),
        in_specs=[rt, rt, rd, rd,
                  pl.BlockSpec((1, D), lambda i: (0, 0)),
                  pl.BlockSpec((D, HID), lambda i: (0, 0)),
                  pl.BlockSpec((1, HID), lambda i: (0, 0)),
                  pl.BlockSpec((1, HID), lambda i: (0, 0))],
        out_specs=[rd, pl.BlockSpec((3, 128), lambda i: (0, 0))],
        out_shape=[
            jax.ShapeDtypeStruct((N, D), jnp.float32),
            jax.ShapeDtypeStruct((3, 128), jnp.float32),
        ],
    )(n0, n1, hB, hA, b, p1w, p1b, p2w)


def _final_b_body(hA_ref, hB_ref, hC_ref, wsum_ref, hout_ref, beta_ref):
    w = wsum_ref[...] * (1.0 / N)
    m = jnp.max(w[:, 0:1])
    ew = jnp.exp(w - m)
    beta = ew / jnp.sum(ew[:, 0:1])
    beta_ref[...] = beta
    bc = beta[:, 0:D]
    hsum = (hA_ref[...] * bc[0:1] + hB_ref[...] * bc[1:2]
            + hC_ref[...] * bc[2:3])
    hout_ref[...] = _l2norm(hsum)


def _tc_final_b(hA, hB, hC, wsum):
    rd = pl.BlockSpec((R, D), lambda i: (i, 0))
    return pl.pallas_call(
        _final_b_body,
        grid=(N // R,),
        in_specs=[rd, rd, rd, pl.BlockSpec((3, 128), lambda i: (0, 0))],
        out_specs=[rd, pl.BlockSpec((3, 128), lambda i: (0, 0))],
        out_shape=[
            jax.ShapeDtypeStruct((N, D), jnp.float32),
            jax.ShapeDtypeStruct((3, 128), jnp.float32),
        ],
    )(hA, hB, hC, wsum)


def kernel(x, edge_index, W0, al0, ar0, b0, W1, al1, ar1, b1,
           W2, al2, ar2, b2, P1W, P1b, P2W):
    src = edge_index[0]
    dst = edge_index[1]
    npad_e = EP - E
    pad_i = jnp.arange(npad_e, dtype=jnp.int32)
    srcp = jnp.concatenate([src, (pad_i * 37) % N])
    dstp = jnp.concatenate([dst, N + (pad_i % NPADROWS)])

    al0r, ar0r = al0[None, :], ar0[None, :]
    al1r, ar1r = al1[None, :], ar1[None, :]
    al2r, ar2r = al2[None, :], ar2[None, :]
    b0r, b1r, b2r = b0[None, :], b1[None, :], b2[None, :]
    p1b = P1b[None, :]
    p2w = P2W[:, 0][None, :]
    pad_er = jnp.full((NPADROWS,), -1e30, jnp.float32)

    def edge_phase(feat, er):
        erp = jnp.concatenate([er[:, 0], pad_er])
        num = _sc_edge(feat, erp, srcp, dstp)
        return num[0, :N, :], num[1, :N, :]

    feat, er = _tc_pre0(x, W0, al0r, ar0r)
    n0, n1 = edge_phase(feat, er)
    feat, er, hA = _tc_pre1(n0, n1, b0r, W1, al1r, ar1r)
    n0, n1 = edge_phase(feat, er)
    feat, er, hB = _tc_pre2(n0, n1, hA, b1r, W2, al2r, ar2r)
    n0, n1 = edge_phase(feat, er)
    hC, wsum = _tc_final_a(n0, n1, hB, hA, b2r, P1W, p1b, p2w)
    h_out, beta = _tc_final_b(hA, hB, hC, wsum)
    return (h_out, beta[:, 0:1])


# speedup vs baseline: 32.0732x; 1.3941x over previous
import functools

import jax
import jax.numpy as jnp
from jax import lax
from jax.experimental import pallas as pl
from jax.experimental.pallas import tpu as pltpu
from jax.experimental.pallas import tpu_sc as plsc

N = 10000
E = 320000
IN_FEATS = 128
D = 64
DT = 128
HID = 16

NC = 2
NS = 16
NW = NC * NS
K = 128
CHUNKS = 80
EP = NW * K * CHUNKS
EPT = K * CHUNKS
GRP = 8
NPADROWS = 112
NPAD = N + NPADROWS
RPT = NPAD // NS
ZCHUNKS = [(0, 128), (128, 128), (256, 128), (384, 128), (512, 120)]

R = 1000


def _sc_edge_body(feat_hbm, erp_hbm, src2_hbm, dst2_hbm,
                  num_out,
                  er_v, sidxg, didxg, rows_v0, rows_v1, ex_v,
                  shared_num, gsem0, gsem1, ssem0, ssem1):
    cid = lax.axis_index("c")
    sid = lax.axis_index("s")
    gwid = cid * NS + sid
    rowbase = gwid * CHUNKS

    pltpu.sync_copy(erp_hbm, er_v)

    zero16 = jnp.zeros((16,), jnp.float32)

    def zero_body(r, _):
        for j in range(DT // 16):
            rows_v0[r, pl.ds(j * 16, 16)] = zero16
        return 0

    lax.fori_loop(0, K, zero_body, 0)

    for (off, sz) in ZCHUNKS:
        pltpu.sync_copy(rows_v0.at[0:sz],
                        shared_num.at[pl.ds(sid * RPT + off, sz)])
    plsc.subcore_barrier()

    lane = lax.iota(jnp.int32, 16)
    col0 = jnp.zeros((16,), jnp.int32)

    def fetch_group(g):
        pltpu.sync_copy(src2_hbm.at[pl.ds(rowbase + g * GRP, GRP)], sidxg)
        pltpu.sync_copy(dst2_hbm.at[pl.ds(rowbase + g * GRP, GRP)], didxg)

    def gather(c, rows_v, gsem):
        pltpu.async_copy(feat_hbm.at[sidxg.at[c % GRP]], rows_v, gsem)

    def compute(rows_v, c):
        didxr = didxg.at[c % GRP]
        for g in range(K // 16):
            didx = didxr[pl.ds(g * 16, 16)]
            elv = plsc.load_gather(rows_v, [g * 16 + lane, col0 + (D + 1)])
            e = elv + plsc.load_gather(er_v, [didx])
            e = jnp.where(e > 0.0, e, 0.2 * e)
            ex_v[pl.ds(g * 16, 16)] = jnp.exp(e)

        def mul_body(k, _):
            bro = plsc.load_gather(ex_v, [col0 + k])
            for j in range(5):
                rows_v[k, pl.ds(j * 16, 16)] = rows_v[k, pl.ds(j * 16, 16)] * bro
            return 0

        lax.fori_loop(0, K, mul_body, 0, unroll=4)

    def scatter(c, rows_v, ssem):
        pltpu.async_copy(rows_v, shared_num.at[didxg.at[c % GRP]], ssem,
                         add=True)

    def swait(c, rows_v, ssem):
        pltpu.make_async_copy(rows_v, shared_num.at[didxg.at[c % GRP]],
                              ssem).wait()

    def gwait(c, rows_v, gsem):
        pltpu.make_async_copy(feat_hbm.at[sidxg.at[c % GRP]], rows_v,
                              gsem).wait()

    fetch_group(0)
    gather(0, rows_v0, gsem0)
    gather(1, rows_v1, gsem1)

    def chunk_pair(i2, _):
        c0 = i2 * 2
        c1 = c0 + 1
        gwait(c0, rows_v0, gsem0)
        compute(rows_v0, c0)
        scatter(c0, rows_v0, ssem0)
        gwait(c1, rows_v1, gsem1)
        compute(rows_v1, c1)
        scatter(c1, rows_v1, ssem1)
        swait(c0, rows_v0, ssem0)
        swait(c1, rows_v1, ssem1)

        @pl.when(c0 + 2 < CHUNKS)
        def _():
            @pl.when((c0 + 2) % GRP == 0)
            def _():
                fetch_group((c0 + 2) // GRP)

            gather(c0 + 2, rows_v0, gsem0)
            gather(c0 + 3, rows_v1, gsem1)

        return 0

    lax.fori_loop(0, CHUNKS // 2, chunk_pair, 0)
    plsc.subcore_barrier()

    for (off, sz) in ZCHUNKS:
        pltpu.sync_copy(shared_num.at[pl.ds(sid * RPT + off, sz)],
                        rows_v0.at[0:sz])
        pltpu.sync_copy(rows_v0.at[0:sz],
                        num_out.at[cid, pl.ds(sid * RPT + off, sz)])


_sc_edge = functools.partial(
    pl.kernel,
    out_type=jax.ShapeDtypeStruct((NC, NPAD, DT), jnp.float32),
    mesh=plsc.VectorSubcoreMesh(core_axis_name="c", subcore_axis_name="s"),
    compiler_params=pltpu.CompilerParams(needs_layout_passes=False),
    scratch_types=[
        pltpu.VMEM((NPAD,), jnp.float32),
        pltpu.VMEM((GRP, K), jnp.int32),
        pltpu.VMEM((GRP, K), jnp.int32),
        pltpu.VMEM((K, DT), jnp.float32),
        pltpu.VMEM((K, DT), jnp.float32),
        pltpu.VMEM((K,), jnp.float32),
        pltpu.VMEM_SHARED((NPAD, DT), jnp.float32),
        pltpu.SemaphoreType.DMA,
        pltpu.SemaphoreType.DMA,
        pltpu.SemaphoreType.DMA,
        pltpu.SemaphoreType.DMA,
    ],
)(_sc_edge_body)


def _l2norm(h):
    n = jnp.sqrt(jnp.sum(h * h, axis=1, keepdims=True))
    return h / jnp.maximum(n, 1e-12)


def _emit(feat, al_ref, ar_ref, feat_ref, er_ref):
    r = feat.shape[0]
    el = jnp.sum(feat * al_ref[...], axis=1, keepdims=True)
    feat_ref[...] = jnp.concatenate(
        [feat, jnp.ones((r, 1), jnp.float32), el,
         jnp.zeros((r, DT - D - 2), jnp.float32)], axis=1)
    er_ref[...] = jnp.sum(feat * ar_ref[...], axis=1, keepdims=True)


def _pre0_body(x_ref, w_ref, al_ref, ar_ref, feat_ref, er_ref):
    hn = _l2norm(x_ref[...])
    feat = jnp.dot(hn, w_ref[...], preferred_element_type=jnp.float32)
    _emit(feat, al_ref, ar_ref, feat_ref, er_ref)


def _tc_pre0(x, w, al, ar):
    return pl.pallas_call(
        _pre0_body,
        grid=(N // R,),
        in_specs=[
            pl.BlockSpec((R, IN_FEATS), lambda i: (i, 0)),
            pl.BlockSpec((IN_FEATS, D), lambda i: (0, 0)),
            pl.BlockSpec((1, D), lambda i: (0, 0)),
            pl.BlockSpec((1, D), lambda i: (0, 0)),
        ],
        out_specs=[
            pl.BlockSpec((R, DT), lambda i: (i, 0)),
            pl.BlockSpec((R, 1), lambda i: (i, 0)),
        ],
        out_shape=[
            jax.ShapeDtypeStruct((N, DT), jnp.float32),
            jax.ShapeDtypeStruct((N, 1), jnp.float32),
        ],
    )(x, w, al, ar)


def _combine(n0_ref, n1_ref, b_ref):
    n0 = n0_ref[...]
    n1 = n1_ref[...]
    s = jnp.maximum(n0[:, D:D + 1] + n1[:, D:D + 1], 1e-12)
    return (n0[:, 0:D] + n1[:, 0:D]) / s + b_ref[...]


def _make_pre_mid(with_prev):
    def body(*refs):
        if with_prev:
            (n0_ref, n1_ref, hprev_ref, b_ref,
             w_ref, al_ref, ar_ref, feat_ref, er_ref, hacc_ref) = refs
        else:
            (n0_ref, n1_ref, b_ref,
             w_ref, al_ref, ar_ref, feat_ref, er_ref, hacc_ref) = refs
        rst = _combine(n0_ref, n1_ref, b_ref)
        hacc = rst + hprev_ref[...] if with_prev else rst
        hacc_ref[...] = hacc
        hn = _l2norm(hacc)
        feat = jnp.dot(hn, w_ref[...], preferred_element_type=jnp.float32)
        _emit(feat, al_ref, ar_ref, feat_ref, er_ref)

    rt = pl.BlockSpec((R, DT), lambda i: (i, 0))
    rd = pl.BlockSpec((R, D), lambda i: (i, 0))
    r1 = pl.BlockSpec((R, 1), lambda i: (i, 0))
    full1d = pl.BlockSpec((1, D), lambda i: (0, 0))
    in_specs = [rt, rt] + ([rd] if with_prev else []) + [
        full1d, pl.BlockSpec((D, D), lambda i: (0, 0)), full1d, full1d]

    def run(*args):
        return pl.pallas_call(
            body,
            grid=(N // R,),
            in_specs=in_specs,
            out_specs=[rt, r1, rd],
            out_shape=[
                jax.ShapeDtypeStruct((N, DT), jnp.float32),
                jax.ShapeDtypeStruct((N, 1), jnp.float32),
                jax.ShapeDtypeStruct((N, D), jnp.float32),
            ],
        )(*args)

    return run


_tc_pre1 = _make_pre_mid(False)
_tc_pre2 = _make_pre_mid(True)


def _score(h, p1w_ref, p1b_ref, p2w_ref):
    t = jnp.tanh(jnp.dot(h, p1w_ref[...], preferred_element_type=jnp.float32)
                 + p1b_ref[...])
    return jnp.sum(t * p2w_ref[...])


def _final_a_body(n0_ref, n1_ref, hB_ref, hA_ref, b_ref,
                  p1w_ref, p1b_ref, p2w_ref, hC_ref, wsum_ref):
    i = pl.program_id(0)
    rst = _combine(n0_ref, n1_ref, b_ref)
    hC = hB_ref[...] + rst
    hC_ref[...] = hC
    w0 = _score(hA_ref[...], p1w_ref, p1b_ref, p2w_ref)
    w1 = _score(hB_ref[...], p1w_ref, p1b_ref, p2w_ref)
    w2 = _score(hC, p1w_ref, p1b_ref, p2w_ref)
    part = jnp.broadcast_to(jnp.stack([w0, w1, w2])[:, None], (3, 128))

    @pl.when(i == 0)
    def _():
        wsum_ref[...] = jnp.zeros_like(wsum_ref)

    wsum_ref[...] += part


def _tc_final_a(n0, n1, hB, hA, b, p1w, p1b, p2w):
    rt = pl.BlockSpec((R, DT), lambda i: (i, 0))
    rd = pl.BlockSpec((R, D), lambda i: (i, 0))
    return pl.pallas_call(
        _final_a_body,
        grid=(N // R,),
        in_specs=[rt, rt, rd, rd,
                  pl.BlockSpec((1, D), lambda i: (0, 0)),
                  pl.BlockSpec((D, HID), lambda i: (0, 0)),
                  pl.BlockSpec((1, HID), lambda i: (0, 0)),
                  pl.BlockSpec((1, HID), lambda i: (0, 0))],
        out_specs=[rd, pl.BlockSpec((3, 128), lambda i: (0, 0))],
        out_shape=[
            jax.ShapeDtypeStruct((N, D), jnp.float32),
            jax.ShapeDtypeStruct((3, 128), jnp.float32),
        ],
    )(n0, n1, hB, hA, b, p1w, p1b, p2w)


def _final_b_body(hA_ref, hB_ref, hC_ref, wsum_ref, hout_ref, beta_ref):
    w = wsum_ref[...] * (1.0 / N)
    m = jnp.max(w[:, 0:1])
    ew = jnp.exp(w - m)
    beta = ew / jnp.sum(ew[:, 0:1])
    beta_ref[...] = beta
    bc = beta[:, 0:D]
    hsum = (hA_ref[...] * bc[0:1] + hB_ref[...] * bc[1:2]
            + hC_ref[...] * bc[2:3])
    hout_ref[...] = _l2norm(hsum)


def _tc_final_b(hA, hB, hC, wsum):
    rd = pl.BlockSpec((R, D), lambda i: (i, 0))
    return pl.pallas_call(
        _final_b_body,
        grid=(N // R,),
        in_specs=[rd, rd, rd, pl.BlockSpec((3, 128), lambda i: (0, 0))],
        out_specs=[rd, pl.BlockSpec((3, 128), lambda i: (0, 0))],
        out_shape=[
            jax.ShapeDtypeStruct((N, D), jnp.float32),
            jax.ShapeDtypeStruct((3, 128), jnp.float32),
        ],
    )(hA, hB, hC, wsum)


def kernel(x, edge_index, W0, al0, ar0, b0, W1, al1, ar1, b1,
           W2, al2, ar2, b2, P1W, P1b, P2W):
    src = edge_index[0]
    dst = edge_index[1]
    npad_e = EP - E
    pad_i = jnp.arange(npad_e, dtype=jnp.int32)
    srcp = jnp.concatenate([src, (pad_i * 37) % N]).reshape(EP // K, K)
    dstp = jnp.concatenate([dst, N + (pad_i % NPADROWS)]).reshape(EP // K, K)

    al0r, ar0r = al0[None, :], ar0[None, :]
    al1r, ar1r = al1[None, :], ar1[None, :]
    al2r, ar2r = al2[None, :], ar2[None, :]
    b0r, b1r, b2r = b0[None, :], b1[None, :], b2[None, :]
    p1b = P1b[None, :]
    p2w = P2W[:, 0][None, :]
    pad_er = jnp.full((NPADROWS,), -1e30, jnp.float32)

    def edge_phase(feat, er):
        erp = jnp.concatenate([er[:, 0], pad_er])
        num = _sc_edge(feat, erp, srcp, dstp)
        return num[0, :N, :], num[1, :N, :]

    feat, er = _tc_pre0(x, W0, al0r, ar0r)
    n0, n1 = edge_phase(feat, er)
    feat, er, hA = _tc_pre1(n0, n1, b0r, W1, al1r, ar1r)
    n0, n1 = edge_phase(feat, er)
    feat, er, hB = _tc_pre2(n0, n1, hA, b1r, W2, al2r, ar2r)
    n0, n1 = edge_phase(feat, er)
    hC, wsum = _tc_final_a(n0, n1, hB, hA, b2r, P1W, p1b, p2w)
    h_out, beta = _tc_final_b(hA, hB, hC, wsum)
    return (h_out, beta[:, 0:1])


# speedup vs baseline: 38.7345x; 1.2077x over previous
import functools

import jax
import jax.numpy as jnp
from jax import lax
from jax.experimental import pallas as pl
from jax.experimental.pallas import tpu as pltpu
from jax.experimental.pallas import tpu_sc as plsc

N = 10000
E = 320000
IN_FEATS = 128
D = 64
DT = 128
HID = 16

NC = 2
NS = 16
NW = NC * NS
K = 128
CHUNKS = 80
EP = NW * K * CHUNKS
EPT = K * CHUNKS
GRP = 16
NPADROWS = 112
NPAD = N + NPADROWS
RPT = NPAD // NS
ZCHUNKS = [(0, 128), (128, 128), (256, 128), (384, 128), (512, 120)]

R = 1000


def _sc_edge_body(feat_hbm, erp_hbm, src2_hbm, dst2_hbm,
                  num_out,
                  er_v, sidxg, didxg, rows_v0, rows_v1, ex_v,
                  shared_num, gsem0, gsem1, ssem0, ssem1):
    cid = lax.axis_index("c")
    sid = lax.axis_index("s")
    gwid = cid * NS + sid
    rowbase = gwid * CHUNKS

    pltpu.sync_copy(erp_hbm, er_v)

    zero16 = jnp.zeros((16,), jnp.float32)

    def zero_body(r, _):
        for j in range(DT // 16):
            rows_v0[r, pl.ds(j * 16, 16)] = zero16
        return 0

    lax.fori_loop(0, K, zero_body, 0)

    for (off, sz) in ZCHUNKS:
        pltpu.sync_copy(rows_v0.at[0:sz],
                        shared_num.at[pl.ds(sid * RPT + off, sz)])
    plsc.subcore_barrier()

    lane = lax.iota(jnp.int32, 16)
    col0 = jnp.zeros((16,), jnp.int32)

    def fetch_group(g):
        pltpu.sync_copy(src2_hbm.at[pl.ds(rowbase + g * GRP, GRP)], sidxg)
        pltpu.sync_copy(dst2_hbm.at[pl.ds(rowbase + g * GRP, GRP)], didxg)

    def gather(c, rows_v, gsem):
        pltpu.async_copy(feat_hbm.at[sidxg.at[c % GRP]], rows_v, gsem)

    def compute(rows_v, c):
        didxr = didxg.at[c % GRP]
        for g in range(K // 16):
            didx = didxr[pl.ds(g * 16, 16)]
            elv = plsc.load_gather(rows_v, [g * 16 + lane, col0 + (D + 1)])
            e = elv + plsc.load_gather(er_v, [didx])
            e = jnp.where(e > 0.0, e, 0.2 * e)
            ex_v[pl.ds(g * 16, 16)] = jnp.exp(e)

        def mul_body(k, _):
            bro = plsc.load_gather(ex_v, [col0 + k])
            for j in range(5):
                rows_v[k, pl.ds(j * 16, 16)] = rows_v[k, pl.ds(j * 16, 16)] * bro
            return 0

        lax.fori_loop(0, K, mul_body, 0, unroll=4)

    def scatter(c, rows_v, ssem):
        pltpu.async_copy(rows_v, shared_num.at[didxg.at[c % GRP]], ssem,
                         add=True)

    def swait(c, rows_v, ssem):
        pltpu.make_async_copy(rows_v, shared_num.at[didxg.at[c % GRP]],
                              ssem).wait()

    def gwait(c, rows_v, gsem):
        pltpu.make_async_copy(feat_hbm.at[sidxg.at[c % GRP]], rows_v,
                              gsem).wait()

    fetch_group(0)
    gather(0, rows_v0, gsem0)
    gather(1, rows_v1, gsem1)

    def chunk_pair(i2, _):
        c0 = i2 * 2
        c1 = c0 + 1
        gwait(c0, rows_v0, gsem0)
        compute(rows_v0, c0)
        scatter(c0, rows_v0, ssem0)
        gwait(c1, rows_v1, gsem1)
        compute(rows_v1, c1)
        scatter(c1, rows_v1, ssem1)
        swait(c0, rows_v0, ssem0)
        last = c0 + 2 >= CHUNKS
        boundary = (c0 + 2) % GRP == 0

        @pl.when(jnp.logical_and(~last, ~boundary))
        def _():
            gather(c0 + 2, rows_v0, gsem0)
            swait(c1, rows_v1, ssem1)
            gather(c0 + 3, rows_v1, gsem1)

        @pl.when(jnp.logical_and(~last, boundary))
        def _():
            swait(c1, rows_v1, ssem1)
            fetch_group((c0 + 2) // GRP)
            gather(c0 + 2, rows_v0, gsem0)
            gather(c0 + 3, rows_v1, gsem1)

        @pl.when(last)
        def _():
            swait(c1, rows_v1, ssem1)

        return 0

    lax.fori_loop(0, CHUNKS // 2, chunk_pair, 0)
    plsc.subcore_barrier()

    for (off, sz) in ZCHUNKS:
        pltpu.sync_copy(shared_num.at[pl.ds(sid * RPT + off, sz)],
                        rows_v0.at[0:sz])
        pltpu.sync_copy(rows_v0.at[0:sz],
                        num_out.at[cid, pl.ds(sid * RPT + off, sz)])


_sc_edge = functools.partial(
    pl.kernel,
    out_type=jax.ShapeDtypeStruct((NC, NPAD, DT), jnp.float32),
    mesh=plsc.VectorSubcoreMesh(core_axis_name="c", subcore_axis_name="s"),
    compiler_params=pltpu.CompilerParams(needs_layout_passes=False),
    scratch_types=[
        pltpu.VMEM((NPAD,), jnp.float32),
        pltpu.VMEM((GRP, K), jnp.int32),
        pltpu.VMEM((GRP, K), jnp.int32),
        pltpu.VMEM((K, DT), jnp.float32),
        pltpu.VMEM((K, DT), jnp.float32),
        pltpu.VMEM((K,), jnp.float32),
        pltpu.VMEM_SHARED((NPAD, DT), jnp.float32),
        pltpu.SemaphoreType.DMA,
        pltpu.SemaphoreType.DMA,
        pltpu.SemaphoreType.DMA,
        pltpu.SemaphoreType.DMA,
    ],
)(_sc_edge_body)


def _l2norm(h):
    n = jnp.sqrt(jnp.sum(h * h, axis=1, keepdims=True))
    return h / jnp.maximum(n, 1e-12)


def _emit(feat, al_ref, ar_ref, feat_ref, er_ref):
    r = feat.shape[0]
    el = jnp.sum(feat * al_ref[...], axis=1, keepdims=True)
    feat_ref[...] = jnp.concatenate(
        [feat, jnp.ones((r, 1), jnp.float32), el,
         jnp.zeros((r, DT - D - 2), jnp.float32)], axis=1)
    er_ref[...] = jnp.sum(feat * ar_ref[...], axis=1, keepdims=True)


def _pre0_body(x_ref, w_ref, al_ref, ar_ref, feat_ref, er_ref):
    hn = _l2norm(x_ref[...])
    feat = jnp.dot(hn, w_ref[...], preferred_element_type=jnp.float32)
    _emit(feat, al_ref, ar_ref, feat_ref, er_ref)


def _tc_pre0(x, w, al, ar):
    return pl.pallas_call(
        _pre0_body,
        grid=(N // R,),
        in_specs=[
            pl.BlockSpec((R, IN_FEATS), lambda i: (i, 0)),
            pl.BlockSpec((IN_FEATS, D), lambda i: (0, 0)),
            pl.BlockSpec((1, D), lambda i: (0, 0)),
            pl.BlockSpec((1, D), lambda i: (0, 0)),
        ],
        out_specs=[
            pl.BlockSpec((R, DT), lambda i: (i, 0)),
            pl.BlockSpec((R, 1), lambda i: (i, 0)),
        ],
        out_shape=[
            jax.ShapeDtypeStruct((N, DT), jnp.float32),
            jax.ShapeDtypeStruct((N, 1), jnp.float32),
        ],
    )(x, w, al, ar)


def _combine(n0_ref, n1_ref, b_ref):
    n0 = n0_ref[...]
    n1 = n1_ref[...]
    s = jnp.maximum(n0[:, D:D + 1] + n1[:, D:D + 1], 1e-12)
    return (n0[:, 0:D] + n1[:, 0:D]) / s + b_ref[...]


def _make_pre_mid(with_prev):
    def body(*refs):
        if with_prev:
            (n0_ref, n1_ref, hprev_ref, b_ref,
             w_ref, al_ref, ar_ref, feat_ref, er_ref, hacc_ref) = refs
        else:
            (n0_ref, n1_ref, b_ref,
             w_ref, al_ref, ar_ref, feat_ref, er_ref, hacc_ref) = refs
        rst = _combine(n0_ref, n1_ref, b_ref)
        hacc = rst + hprev_ref[...] if with_prev else rst
        hacc_ref[...] = hacc
        hn = _l2norm(hacc)
        feat = jnp.dot(hn, w_ref[...], preferred_element_type=jnp.float32)
        _emit(feat, al_ref, ar_ref, feat_ref, er_ref)

    rt = pl.BlockSpec((R, DT), lambda i: (i, 0))
    rd = pl.BlockSpec((R, D), lambda i: (i, 0))
    r1 = pl.BlockSpec((R, 1), lambda i: (i, 0))
    full1d = pl.BlockSpec((1, D), lambda i: (0, 0))
    in_specs = [rt, rt] + ([rd] if with_prev else []) + [
        full1d, pl.BlockSpec((D, D), lambda i: (0, 0)), full1d, full1d]

    def run(*args):
        return pl.pallas_call(
            body,
            grid=(N // R,),
            in_specs=in_specs,
            out_specs=[rt, r1, rd],
            out_shape=[
                jax.ShapeDtypeStruct((N, DT), jnp.float32),
                jax.ShapeDtypeStruct((N, 1), jnp.float32),
                jax.ShapeDtypeStruct((N, D), jnp.float32),
            ],
        )(*args)

    return run


_tc_pre1 = _make_pre_mid(False)
_tc_pre2 = _make_pre_mid(True)


def _score(h, p1w_ref, p1b_ref, p2w_ref):
    t = jnp.tanh(jnp.dot(h, p1w_ref[...], preferred_element_type=jnp.float32)
                 + p1b_ref[...])
    return jnp.sum(t * p2w_ref[...])


def _final_a_body(n0_ref, n1_ref, hB_ref, hA_ref, b_ref,
                  p1w_ref, p1b_ref, p2w_ref, hC_ref, wsum_ref):
    i = pl.program_id(0)
    rst = _combine(n0_ref, n1_ref, b_ref)
    hC = hB_ref[...] + rst
    hC_ref[...] = hC
    w0 = _score(hA_ref[...], p1w_ref, p1b_ref, p2w_ref)
    w1 = _score(hB_ref[...], p1w_ref, p1b_ref, p2w_ref)
    w2 = _score(hC, p1w_ref, p1b_ref, p2w_ref)
    part = jnp.broadcast_to(jnp.stack([w0, w1, w2])[:, None], (3, 128))

    @pl.when(i == 0)
    def _():
        wsum_ref[...] = jnp.zeros_like(wsum_ref)

    wsum_ref[...] += part


def _tc_final_a(n0, n1, hB, hA, b, p1w, p1b, p2w):
    rt = pl.BlockSpec((R, DT), lambda i: (i, 0))
    rd = pl.BlockSpec((R, D), lambda i: (i, 0))
    return pl.pallas_call(
        _final_a_body,
        grid=(N // R,),
        in_specs=[rt, rt, rd, rd,
                  pl.BlockSpec((1, D), lambda i: (0, 0)),
                  pl.BlockSpec((D, HID), lambda i: (0, 0)),
                  pl.BlockSpec((1, HID), lambda i: (0, 0)),
                  pl.BlockSpec((1, HID), lambda i: (0, 0))],
        out_specs=[rd, pl.BlockSpec((3, 128), lambda i: (0, 0))],
        out_shape=[
            jax.ShapeDtypeStruct((N, D), jnp.float32),
            jax.ShapeDtypeStruct((3, 128), jnp.float32),
        ],
    )(n0, n1, hB, hA, b, p1w, p1b, p2w)


def _final_b_body(hA_ref, hB_ref, hC_ref, wsum_ref, hout_ref, beta_ref):
    w = wsum_ref[...] * (1.0 / N)
    m = jnp.max(w[:, 0:1])
    ew = jnp.exp(w - m)
    beta = ew / jnp.sum(ew[:, 0:1])
    beta_ref[...] = beta
    bc = beta[:, 0:D]
    hsum = (hA_ref[...] * bc[0:1] + hB_ref[...] * bc[1:2]
            + hC_ref[...] * bc[2:3])
    hout_ref[...] = _l2norm(hsum)


def _tc_final_b(hA, hB, hC, wsum):
    rd = pl.BlockSpec((R, D), lambda i: (i, 0))
    return pl.pallas_call(
        _final_b_body,
        grid=(N // R,),
        in_specs=[rd, rd, rd, pl.BlockSpec((3, 128), lambda i: (0, 0))],
        out_specs=[rd, pl.BlockSpec((3, 128), lambda i: (0, 0))],
        out_shape=[
            jax.ShapeDtypeStruct((N, D), jnp.float32),
            jax.ShapeDtypeStruct((3, 128), jnp.float32),
        ],
    )(hA, hB, hC, wsum)


def kernel(x, edge_index, W0, al0, ar0, b0, W1, al1, ar1, b1,
           W2, al2, ar2, b2, P1W, P1b, P2W):
    src = edge_index[0]
    dst = edge_index[1]
    npad_e = EP - E
    pad_i = jnp.arange(npad_e, dtype=jnp.int32)
    srcp = jnp.concatenate([src, (pad_i * 37) % N]).reshape(EP // K, K)
    dstp = jnp.concatenate([dst, N + (pad_i % NPADROWS)]).reshape(EP // K, K)

    al0r, ar0r = al0[None, :], ar0[None, :]
    al1r, ar1r = al1[None, :], ar1[None, :]
    al2r, ar2r = al2[None, :], ar2[None, :]
    b0r, b1r, b2r = b0[None, :], b1[None, :], b2[None, :]
    p1b = P1b[None, :]
    p2w = P2W[:, 0][None, :]
    pad_er = jnp.full((NPADROWS,), -1e30, jnp.float32)

    def edge_phase(feat, er):
        erp = jnp.concatenate([er[:, 0], pad_er])
        num = _sc_edge(feat, erp, srcp, dstp)
        return num[0, :N, :], num[1, :N, :]

    feat, er = _tc_pre0(x, W0, al0r, ar0r)
    n0, n1 = edge_phase(feat, er)
    feat, er, hA = _tc_pre1(n0, n1, b0r, W1, al1r, ar1r)
    n0, n1 = edge_phase(feat, er)
    feat, er, hB = _tc_pre2(n0, n1, hA, b1r, W2, al2r, ar2r)
    n0, n1 = edge_phase(feat, er)
    hC, wsum = _tc_final_a(n0, n1, hB, hA, b2r, P1W, p1b, p2w)
    h_out, beta = _tc_final_b(hA, hB, hC, wsum)
    return (h_out, beta[:, 0:1])


# speedup vs baseline: 38.7582x; 1.0006x over previous
import functools

import jax
import jax.numpy as jnp
from jax import lax
from jax.experimental import pallas as pl
from jax.experimental.pallas import tpu as pltpu
from jax.experimental.pallas import tpu_sc as plsc

N = 10000
E = 320000
IN_FEATS = 128
D = 64
DT = 128
HID = 16

NC = 2
NS = 16
NW = NC * NS
K = 128
CHUNKS = 80
EP = NW * K * CHUNKS
EPT = K * CHUNKS
GRP = 16
NPADROWS = 112
NPAD = N + NPADROWS
RPT = NPAD // NS
ZCHUNKS = [(0, 128), (128, 128), (256, 128), (384, 128), (512, 120)]

R = 1000


def _sc_edge_body(feat_hbm, erp_hbm, src2_hbm, dst2_hbm,
                  num_out,
                  er_v, sidxg, didxg, rows_v0, rows_v1, ex_v,
                  shared_num, gsem0, gsem1, ssem0, ssem1):
    cid = lax.axis_index("c")
    sid = lax.axis_index("s")
    gwid = cid * NS + sid
    rowbase = gwid * CHUNKS

    pltpu.sync_copy(erp_hbm, er_v)

    zero16 = jnp.zeros((16,), jnp.float32)

    def zero_body(r, _):
        for j in range(DT // 16):
            rows_v0[r, pl.ds(j * 16, 16)] = zero16
        return 0

    lax.fori_loop(0, K, zero_body, 0)

    for (off, sz) in ZCHUNKS:
        pltpu.sync_copy(rows_v0.at[0:sz],
                        shared_num.at[pl.ds(sid * RPT + off, sz)])
    plsc.subcore_barrier()

    lane = lax.iota(jnp.int32, 16)
    col0 = jnp.zeros((16,), jnp.int32)

    def fetch_group(g):
        pltpu.sync_copy(src2_hbm.at[pl.ds(rowbase + g * GRP, GRP)], sidxg)
        pltpu.sync_copy(dst2_hbm.at[pl.ds(rowbase + g * GRP, GRP)], didxg)

    def gather(c, rows_v, gsem):
        pltpu.async_copy(feat_hbm.at[sidxg.at[c % GRP]], rows_v, gsem)

    def compute(rows_v, c):
        didxr = didxg.at[c % GRP]
        for g in range(K // 16):
            didx = didxr[pl.ds(g * 16, 16)]
            elv = plsc.load_gather(rows_v, [g * 16 + lane, col0 + (D + 1)])
            e = elv + plsc.load_gather(er_v, [didx])
            e = jnp.where(e > 0.0, e, 0.2 * e)
            ex_v[pl.ds(g * 16, 16)] = jnp.exp(e)

        def mul_body(k, _):
            bro = plsc.load_gather(ex_v, [col0 + k])
            for j in range(5):
                rows_v[k, pl.ds(j * 16, 16)] = rows_v[k, pl.ds(j * 16, 16)] * bro
            return 0

        lax.fori_loop(0, K, mul_body, 0, unroll=8)

    def scatter(c, rows_v, ssem):
        pltpu.async_copy(rows_v, shared_num.at[didxg.at[c % GRP]], ssem,
                         add=True)

    def swait(c, rows_v, ssem):
        pltpu.make_async_copy(rows_v, shared_num.at[didxg.at[c % GRP]],
                              ssem).wait()

    def gwait(c, rows_v, gsem):
        pltpu.make_async_copy(feat_hbm.at[sidxg.at[c % GRP]], rows_v,
                              gsem).wait()

    fetch_group(0)
    gather(0, rows_v0, gsem0)
    gather(1, rows_v1, gsem1)

    def chunk_pair(i2, _):
        c0 = i2 * 2
        c1 = c0 + 1
        gwait(c0, rows_v0, gsem0)
        compute(rows_v0, c0)
        scatter(c0, rows_v0, ssem0)
        gwait(c1, rows_v1, gsem1)
        compute(rows_v1, c1)
        scatter(c1, rows_v1, ssem1)
        swait(c0, rows_v0, ssem0)
        last = c0 + 2 >= CHUNKS
        boundary = (c0 + 2) % GRP == 0

        @pl.when(jnp.logical_and(~last, ~boundary))
        def _():
            gather(c0 + 2, rows_v0, gsem0)
            swait(c1, rows_v1, ssem1)
            gather(c0 + 3, rows_v1, gsem1)

        @pl.when(jnp.logical_and(~last, boundary))
        def _():
            swait(c1, rows_v1, ssem1)
            fetch_group((c0 + 2) // GRP)
            gather(c0 + 2, rows_v0, gsem0)
            gather(c0 + 3, rows_v1, gsem1)

        @pl.when(last)
        def _():
            swait(c1, rows_v1, ssem1)

        return 0

    lax.fori_loop(0, CHUNKS // 2, chunk_pair, 0)
    plsc.subcore_barrier()

    for (off, sz) in ZCHUNKS:
        pltpu.sync_copy(shared_num.at[pl.ds(sid * RPT + off, sz)],
                        rows_v0.at[0:sz])
        pltpu.sync_copy(rows_v0.at[0:sz],
                        num_out.at[cid, pl.ds(sid * RPT + off, sz)])


_sc_edge = functools.partial(
    pl.kernel,
    out_type=jax.ShapeDtypeStruct((NC, NPAD, DT), jnp.float32),
    mesh=plsc.VectorSubcoreMesh(core_axis_name="c", subcore_axis_name="s"),
    compiler_params=pltpu.CompilerParams(needs_layout_passes=False),
    scratch_types=[
        pltpu.VMEM((NPAD,), jnp.float32),
        pltpu.VMEM((GRP, K), jnp.int32),
        pltpu.VMEM((GRP, K), jnp.int32),
        pltpu.VMEM((K, DT), jnp.float32),
        pltpu.VMEM((K, DT), jnp.float32),
        pltpu.VMEM((K,), jnp.float32),
        pltpu.VMEM_SHARED((NPAD, DT), jnp.float32),
        pltpu.SemaphoreType.DMA,
        pltpu.SemaphoreType.DMA,
        pltpu.SemaphoreType.DMA,
        pltpu.SemaphoreType.DMA,
    ],
)(_sc_edge_body)


def _l2norm(h):
    n = jnp.sqrt(jnp.sum(h * h, axis=1, keepdims=True))
    return h / jnp.maximum(n, 1e-12)


def _emit(feat, al_ref, ar_ref, feat_ref, er_ref):
    r = feat.shape[0]
    el = jnp.sum(feat * al_ref[...], axis=1, keepdims=True)
    feat_ref[...] = jnp.concatenate(
        [feat, jnp.ones((r, 1), jnp.float32), el,
         jnp.zeros((r, DT - D - 2), jnp.float32)], axis=1)
    er_ref[...] = jnp.sum(feat * ar_ref[...], axis=1, keepdims=True)


def _pre0_body(x_ref, w_ref, al_ref, ar_ref, feat_ref, er_ref):
    hn = _l2norm(x_ref[...])
    feat = jnp.dot(hn, w_ref[...], preferred_element_type=jnp.float32)
    _emit(feat, al_ref, ar_ref, feat_ref, er_ref)


def _tc_pre0(x, w, al, ar):
    return pl.pallas_call(
        _pre0_body,
        grid=(N // R,),
        in_specs=[
            pl.BlockSpec((R, IN_FEATS), lambda i: (i, 0)),
            pl.BlockSpec((IN_FEATS, D), lambda i: (0, 0)),
            pl.BlockSpec((1, D), lambda i: (0, 0)),
            pl.BlockSpec((1, D), lambda i: (0, 0)),
        ],
        out_specs=[
            pl.BlockSpec((R, DT), lambda i: (i, 0)),
            pl.BlockSpec((R, 1), lambda i: (i, 0)),
        ],
        out_shape=[
            jax.ShapeDtypeStruct((N, DT), jnp.float32),
            jax.ShapeDtypeStruct((N, 1), jnp.float32),
        ],
    )(x, w, al, ar)


def _combine(n0_ref, n1_ref, b_ref):
    n0 = n0_ref[...]
    n1 = n1_ref[...]
    s = jnp.maximum(n0[:, D:D + 1] + n1[:, D:D + 1], 1e-12)
    return (n0[:, 0:D] + n1[:, 0:D]) / s + b_ref[...]


def _make_pre_mid(with_prev):
    def body(*refs):
        if with_prev:
            (n0_ref, n1_ref, hprev_ref, b_ref,
             w_ref, al_ref, ar_ref, feat_ref, er_ref, hacc_ref) = refs
        else:
            (n0_ref, n1_ref, b_ref,
             w_ref, al_ref, ar_ref, feat_ref, er_ref, hacc_ref) = refs
        rst = _combine(n0_ref, n1_ref, b_ref)
        hacc = rst + hprev_ref[...] if with_prev else rst
        hacc_ref[...] = hacc
        hn = _l2norm(hacc)
        feat = jnp.dot(hn, w_ref[...], preferred_element_type=jnp.float32)
        _emit(feat, al_ref, ar_ref, feat_ref, er_ref)

    rt = pl.BlockSpec((R, DT), lambda i: (i, 0))
    rd = pl.BlockSpec((R, D), lambda i: (i, 0))
    r1 = pl.BlockSpec((R, 1), lambda i: (i, 0))
    full1d = pl.BlockSpec((1, D), lambda i: (0, 0))
    in_specs = [rt, rt] + ([rd] if with_prev else []) + [
        full1d, pl.BlockSpec((D, D), lambda i: (0, 0)), full1d, full1d]

    def run(*args):
        return pl.pallas_call(
            body,
            grid=(N // R,),
            in_specs=in_specs,
            out_specs=[rt, r1, rd],
            out_shape=[
                jax.ShapeDtypeStruct((N, DT), jnp.float32),
                jax.ShapeDtypeStruct((N, 1), jnp.float32),
                jax.ShapeDtypeStruct((N, D), jnp.float32),
            ],
        )(*args)

    return run


_tc_pre1 = _make_pre_mid(False)
_tc_pre2 = _make_pre_mid(True)


def _score(h, p1w_ref, p1b_ref, p2w_ref):
    t = jnp.tanh(jnp.dot(h, p1w_ref[...], preferred_element_type=jnp.float32)
                 + p1b_ref[...])
    return jnp.sum(t * p2w_ref[...])


def _final_a_body(n0_ref, n1_ref, hB_ref, hA_ref, b_ref,
                  p1w_ref, p1b_ref, p2w_ref, hC_ref, wsum_ref):
    i = pl.program_id(0)
    rst = _combine(n0_ref, n1_ref, b_ref)
    hC = hB_ref[...] + rst
    hC_ref[...] = hC
    w0 = _score(hA_ref[...], p1w_ref, p1b_ref, p2w_ref)
    w1 = _score(hB_ref[...], p1w_ref, p1b_ref, p2w_ref)
    w2 = _score(hC, p1w_ref, p1b_ref, p2w_ref)
    part = jnp.broadcast_to(jnp.stack([w0, w1, w2])[:, None], (3, 128))

    @pl.when(i == 0)
    def _():
        wsum_ref[...] = jnp.zeros_like(wsum_ref)

    wsum_ref[...] += part


def _tc_final_a(n0, n1, hB, hA, b, p1w, p1b, p2w):
    rt = pl.BlockSpec((R, DT), lambda i: (i, 0))
    rd = pl.BlockSpec((R, D), lambda i: (i, 0))
    return pl.pallas_call(
        _final_a_body,
        grid=(N // R,),
        in_specs=[rt, rt, rd, rd,
                  pl.BlockSpec((1, D), lambda i: (0, 0)),
                  pl.BlockSpec((D, HID), lambda i: (0, 0)),
                  pl.BlockSpec((1, HID), lambda i: (0, 0)),
                  pl.BlockSpec((1, HID), lambda i: (0, 0))],
        out_specs=[rd, pl.BlockSpec((3, 128), lambda i: (0, 0))],
        out_shape=[
            jax.ShapeDtypeStruct((N, D), jnp.float32),
            jax.ShapeDtypeStruct((3, 128), jnp.float32),
        ],
    )(n0, n1, hB, hA, b, p1w, p1b, p2w)


def _final_b_body(hA_ref, hB_ref, hC_ref, wsum_ref, hout_ref, beta_ref):
    w = wsum_ref[...] * (1.0 / N)
    m = jnp.max(w[:, 0:1])
    ew = jnp.exp(w - m)
    beta = ew / jnp.sum(ew[:, 0:1])
    beta_ref[...] = beta
    bc = beta[:, 0:D]
    hsum = (hA_ref[...] * bc[0:1] + hB_ref[...] * bc[1:2]
            + hC_ref[...] * bc[2:3])
    hout_ref[...] = _l2norm(hsum)


def _tc_final_b(hA, hB, hC, wsum):
    rd = pl.BlockSpec((R, D), lambda i: (i, 0))
    return pl.pallas_call(
        _final_b_body,
        grid=(N // R,),
        in_specs=[rd, rd, rd, pl.BlockSpec((3, 128), lambda i: (0, 0))],
        out_specs=[rd, pl.BlockSpec((3, 128), lambda i: (0, 0))],
        out_shape=[
            jax.ShapeDtypeStruct((N, D), jnp.float32),
            jax.ShapeDtypeStruct((3, 128), jnp.float32),
        ],
    )(hA, hB, hC, wsum)


def kernel(x, edge_index, W0, al0, ar0, b0, W1, al1, ar1, b1,
           W2, al2, ar2, b2, P1W, P1b, P2W):
    src = edge_index[0]
    dst = edge_index[1]
    npad_e = EP - E
    pad_i = jnp.arange(npad_e, dtype=jnp.int32)
    srcp = jnp.concatenate([src, (pad_i * 37) % N]).reshape(EP // K, K)
    dstp = jnp.concatenate([dst, N + (pad_i % NPADROWS)]).reshape(EP // K, K)

    al0r, ar0r = al0[None, :], ar0[None, :]
    al1r, ar1r = al1[None, :], ar1[None, :]
    al2r, ar2r = al2[None, :], ar2[None, :]
    b0r, b1r, b2r = b0[None, :], b1[None, :], b2[None, :]
    p1b = P1b[None, :]
    p2w = P2W[:, 0][None, :]
    pad_er = jnp.full((NPADROWS,), -1e30, jnp.float32)

    def edge_phase(feat, er):
        erp = jnp.concatenate([er[:, 0], pad_er])
        num = _sc_edge(feat, erp, srcp, dstp)
        return num[0, :N, :], num[1, :N, :]

    feat, er = _tc_pre0(x, W0, al0r, ar0r)
    n0, n1 = edge_phase(feat, er)
    feat, er, hA = _tc_pre1(n0, n1, b0r, W1, al1r, ar1r)
    n0, n1 = edge_phase(feat, er)
    feat, er, hB = _tc_pre2(n0, n1, hA, b1r, W2, al2r, ar2r)
    n0, n1 = edge_phase(feat, er)
    hC, wsum = _tc_final_a(n0, n1, hB, hA, b2r, P1W, p1b, p2w)
    h_out, beta = _tc_final_b(hA, hB, hC, wsum)
    return (h_out, beta[:, 0:1])


# speedup vs baseline: 39.9137x; 1.0298x over previous
import functools

import jax
import jax.numpy as jnp
from jax import lax
from jax.experimental import pallas as pl
from jax.experimental.pallas import tpu as pltpu
from jax.experimental.pallas import tpu_sc as plsc

N = 10000
E = 320000
IN_FEATS = 128
D = 64
DT = 128
HID = 16

NC = 2
NS = 16
NW = NC * NS
K = 64
CHUNKS = 160
EP = NW * K * CHUNKS
EPT = K * CHUNKS
GRP = 8
NGRP = CHUNKS // GRP
NPADROWS = 112
NPAD = N + NPADROWS
RPT = NPAD // NS
ZCHUNKS = [(i * 64, 64) for i in range(9)] + [(576, 56)]

R = 1000


def _sc_edge_body(feat_hbm, erp_hbm, src2_hbm, dst2_hbm,
                  num_out,
                  er_v, sidxg, didxg, rows_v0, rows_v1, rows_v2, rows_v3,
                  ex_v, shared_num,
                  gsem0, gsem1, gsem2, gsem3, ssem0, ssem1, ssem2, ssem3):
    cid = lax.axis_index("c")
    sid = lax.axis_index("s")
    gwid = cid * NS + sid
    rowbase = gwid * CHUNKS
    bufs = [rows_v0, rows_v1, rows_v2, rows_v3]
    gsems = [gsem0, gsem1, gsem2, gsem3]
    ssems = [ssem0, ssem1, ssem2, ssem3]

    pltpu.sync_copy(erp_hbm, er_v)

    zero16 = jnp.zeros((16,), jnp.float32)

    def zero_body(r, _):
        for j in range(DT // 16):
            rows_v0[r, pl.ds(j * 16, 16)] = zero16
        return 0

    lax.fori_loop(0, K, zero_body, 0)

    for (off, sz) in ZCHUNKS:
        pltpu.sync_copy(rows_v0.at[0:sz],
                        shared_num.at[pl.ds(sid * RPT + off, sz)])
    plsc.subcore_barrier()

    lane = lax.iota(jnp.int32, 16)
    col0 = jnp.zeros((16,), jnp.int32)

    def fetch_group(g):
        p = g % 2
        pltpu.sync_copy(src2_hbm.at[pl.ds(rowbase + g * GRP, GRP)],
                        sidxg.at[p])
        pltpu.sync_copy(dst2_hbm.at[pl.ds(rowbase + g * GRP, GRP)],
                        didxg.at[p])

    def sidx_ref(c):
        return sidxg.at[(c // GRP) % 2, c % GRP]

    def didx_ref(c):
        return didxg.at[(c // GRP) % 2, c % GRP]

    def gather(c, rows_v, gsem):
        pltpu.async_copy(feat_hbm.at[sidx_ref(c)], rows_v, gsem)

    def compute(rows_v, c):
        didxr = didx_ref(c)
        for g in range(K // 16):
            didx = didxr[pl.ds(g * 16, 16)]
            elv = plsc.load_gather(rows_v, [g * 16 + lane, col0 + (D + 1)])
            e = elv + plsc.load_gather(er_v, [didx])
            e = jnp.where(e > 0.0, e, 0.2 * e)
            ex_v[pl.ds(g * 16, 16)] = jnp.exp(e)

        def mul_body(k, _):
            bro = plsc.load_gather(ex_v, [col0 + k])
            for j in range(5):
                rows_v[k, pl.ds(j * 16, 16)] = rows_v[k, pl.ds(j * 16, 16)] * bro
            return 0

        lax.fori_loop(0, K, mul_body, 0, unroll=8)

    def scatter(c, rows_v, ssem):
        pltpu.async_copy(rows_v, shared_num.at[didx_ref(c)], ssem, add=True)

    def swait(c, rows_v, ssem):
        pltpu.make_async_copy(rows_v, shared_num.at[didx_ref(c)], ssem).wait()

    def gwait(c, rows_v, gsem):
        pltpu.make_async_copy(feat_hbm.at[sidx_ref(c)], rows_v, gsem).wait()

    fetch_group(0)
    gather(0, rows_v0, gsem0)
    gather(1, rows_v1, gsem1)

    def quad(i4, _):
        for q in range(4):
            c = i4 * 4 + q
            bq = bufs[q]
            bn = bufs[(q + 2) % 4]
            gwait(c, bq, gsems[q])
            compute(bq, c)
            scatter(c, bq, ssems[q])

            @pl.when(jnp.logical_and(c % GRP == 4, c < (NGRP - 1) * GRP))
            def _():
                fetch_group(c // GRP + 1)

            @pl.when(c + 2 < CHUNKS)
            def _():
                @pl.when(c >= 2)
                def _():
                    swait(c - 2, bn, ssems[(q + 2) % 4])

                gather(c + 2, bn, gsems[(q + 2) % 4])

        return 0

    lax.fori_loop(0, CHUNKS // 4, quad, 0)
    for c in range(CHUNKS - 4, CHUNKS):
        swait(c, bufs[c % 4], ssems[c % 4])
    plsc.subcore_barrier()

    for (off, sz) in ZCHUNKS:
        pltpu.sync_copy(shared_num.at[pl.ds(sid * RPT + off, sz)],
                        rows_v0.at[0:sz])
        pltpu.sync_copy(rows_v0.at[0:sz],
                        num_out.at[cid, pl.ds(sid * RPT + off, sz)])


_sc_edge = functools.partial(
    pl.kernel,
    out_type=jax.ShapeDtypeStruct((NC, NPAD, DT), jnp.float32),
    mesh=plsc.VectorSubcoreMesh(core_axis_name="c", subcore_axis_name="s"),
    compiler_params=pltpu.CompilerParams(needs_layout_passes=False),
    scratch_types=[
        pltpu.VMEM((NPAD,), jnp.float32),
        pltpu.VMEM((2, GRP, K), jnp.int32),
        pltpu.VMEM((2, GRP, K), jnp.int32),
        pltpu.VMEM((K, DT), jnp.float32),
        pltpu.VMEM((K, DT), jnp.float32),
        pltpu.VMEM((K, DT), jnp.float32),
        pltpu.VMEM((K, DT), jnp.float32),
        pltpu.VMEM((K,), jnp.float32),
        pltpu.VMEM_SHARED((NPAD, DT), jnp.float32),
        pltpu.SemaphoreType.DMA,
        pltpu.SemaphoreType.DMA,
        pltpu.SemaphoreType.DMA,
        pltpu.SemaphoreType.DMA,
        pltpu.SemaphoreType.DMA,
        pltpu.SemaphoreType.DMA,
        pltpu.SemaphoreType.DMA,
        pltpu.SemaphoreType.DMA,
    ],
)(_sc_edge_body)


def _l2norm(h):
    n = jnp.sqrt(jnp.sum(h * h, axis=1, keepdims=True))
    return h / jnp.maximum(n, 1e-12)


def _emit(feat, al_ref, ar_ref, feat_ref, er_ref):
    r = feat.shape[0]
    el = jnp.sum(feat * al_ref[...], axis=1, keepdims=True)
    feat_ref[...] = jnp.concatenate(
        [feat, jnp.ones((r, 1), jnp.float32), el,
         jnp.zeros((r, DT - D - 2), jnp.float32)], axis=1)
    er_ref[...] = jnp.sum(feat * ar_ref[...], axis=1, keepdims=True)


def _pre0_body(x_ref, w_ref, al_ref, ar_ref, feat_ref, er_ref):
    hn = _l2norm(x_ref[...])
    feat = jnp.dot(hn, w_ref[...], preferred_element_type=jnp.float32)
    _emit(feat, al_ref, ar_ref, feat_ref, er_ref)


def _tc_pre0(x, w, al, ar):
    return pl.pallas_call(
        _pre0_body,
        grid=(N // R,),
        in_specs=[
            pl.BlockSpec((R, IN_FEATS), lambda i: (i, 0)),
            pl.BlockSpec((IN_FEATS, D), lambda i: (0, 0)),
            pl.BlockSpec((1, D), lambda i: (0, 0)),
            pl.BlockSpec((1, D), lambda i: (0, 0)),
        ],
        out_specs=[
            pl.BlockSpec((R, DT), lambda i: (i, 0)),
            pl.BlockSpec((R, 1), lambda i: (i, 0)),
        ],
        out_shape=[
            jax.ShapeDtypeStruct((N, DT), jnp.float32),
            jax.ShapeDtypeStruct((N, 1), jnp.float32),
        ],
    )(x, w, al, ar)


def _combine(n0_ref, n1_ref, b_ref):
    n0 = n0_ref[...]
    n1 = n1_ref[...]
    s = jnp.maximum(n0[:, D:D + 1] + n1[:, D:D + 1], 1e-12)
    return (n0[:, 0:D] + n1[:, 0:D]) / s + b_ref[...]


def _make_pre_mid(with_prev):
    def body(*refs):
        if with_prev:
            (n0_ref, n1_ref, hprev_ref, b_ref,
             w_ref, al_ref, ar_ref, feat_ref, er_ref, hacc_ref) = refs
        else:
            (n0_ref, n1_ref, b_ref,
             w_ref, al_ref, ar_ref, feat_ref, er_ref, hacc_ref) = refs
        rst = _combine(n0_ref, n1_ref, b_ref)
        hacc = rst + hprev_ref[...] if with_prev else rst
        hacc_ref[...] = hacc
        hn = _l2norm(hacc)
        feat = jnp.dot(hn, w_ref[...], preferred_element_type=jnp.float32)
        _emit(feat, al_ref, ar_ref, feat_ref, er_ref)

    rt = pl.BlockSpec((R, DT), lambda i: (i, 0))
    rd = pl.BlockSpec((R, D), lambda i: (i, 0))
    r1 = pl.BlockSpec((R, 1), lambda i: (i, 0))
    full1d = pl.BlockSpec((1, D), lambda i: (0, 0))
    in_specs = [rt, rt] + ([rd] if with_prev else []) + [
        full1d, pl.BlockSpec((D, D), lambda i: (0, 0)), full1d, full1d]

    def run(*args):
        return pl.pallas_call(
            body,
            grid=(N // R,),
            in_specs=in_specs,
            out_specs=[rt, r1, rd],
            out_shape=[
                jax.ShapeDtypeStruct((N, DT), jnp.float32),
                jax.ShapeDtypeStruct((N, 1), jnp.float32),
                jax.ShapeDtypeStruct((N, D), jnp.float32),
            ],
        )(*args)

    return run


_tc_pre1 = _make_pre_mid(False)
_tc_pre2 = _make_pre_mid(True)


def _score(h, p1w_ref, p1b_ref, p2w_ref):
    t = jnp.tanh(jnp.dot(h, p1w_ref[...], preferred_element_type=jnp.float32)
                 + p1b_ref[...])
    return jnp.sum(t * p2w_ref[...])


def _final_a_body(n0_ref, n1_ref, hB_ref, hA_ref, b_ref,
                  p1w_ref, p1b_ref, p2w_ref, hC_ref, wsum_ref):
    i = pl.program_id(0)
    rst = _combine(n0_ref, n1_ref, b_ref)
    hC = hB_ref[...] + rst
    hC_ref[...] = hC
    w0 = _score(hA_ref[...], p1w_ref, p1b_ref, p2w_ref)
    w1 = _score(hB_ref[...], p1w_ref, p1b_ref, p2w_ref)
    w2 = _score(hC, p1w_ref, p1b_ref, p2w_ref)
    part = jnp.broadcast_to(jnp.stack([w0, w1, w2])[:, None], (3, 128))

    @pl.when(i == 0)
    def _():
        wsum_ref[...] = jnp.zeros_like(wsum_ref)

    wsum_ref[...] += part


def _tc_final_a(n0, n1, hB, hA, b, p1w, p1b, p2w):
    rt = pl.BlockSpec((R, DT), lambda i: (i, 0))
    rd = pl.BlockSpec((R, D), lambda i: (i, 0))
    return pl.pallas_call(
        _final_a_body,
        grid=(N // R,),
        in_specs=[rt, rt, rd, rd,
                  pl.BlockSpec((1, D), lambda i: (0, 0)),
                  pl.BlockSpec((D, HID), lambda i: (0, 0)),
                  pl.BlockSpec((1, HID), lambda i: (0, 0)),
                  pl.BlockSpec((1, HID), lambda i: (0, 0))],
        out_specs=[rd, pl.BlockSpec((3, 128), lambda i: (0, 0))],
        out_shape=[
            jax.ShapeDtypeStruct((N, D), jnp.float32),
            jax.ShapeDtypeStruct((3, 128), jnp.float32),
        ],
    )(n0, n1, hB, hA, b, p1w, p1b, p2w)


def _final_b_body(hA_ref, hB_ref, hC_ref, wsum_ref, hout_ref, beta_ref):
    w = wsum_ref[...] * (1.0 / N)
    m = jnp.max(w[:, 0:1])
    ew = jnp.exp(w - m)
    beta = ew / jnp.sum(ew[:, 0:1])
    beta_ref[...] = beta
    bc = beta[:, 0:D]
    hsum = (hA_ref[...] * bc[0:1] + hB_ref[...] * bc[1:2]
            + hC_ref[...] * bc[2:3])
    hout_ref[...] = _l2norm(hsum)


def _tc_final_b(hA, hB, hC, wsum):
    rd = pl.BlockSpec((R, D), lambda i: (i, 0))
    return pl.pallas_call(
        _final_b_body,
        grid=(N // R,),
        in_specs=[rd, rd, rd, pl.BlockSpec((3, 128), lambda i: (0, 0))],
        out_specs=[rd, pl.BlockSpec((3, 128), lambda i: (0, 0))],
        out_shape=[
            jax.ShapeDtypeStruct((N, D), jnp.float32),
            jax.ShapeDtypeStruct((3, 128), jnp.float32),
        ],
    )(hA, hB, hC, wsum)


def kernel(x, edge_index, W0, al0, ar0, b0, W1, al1, ar1, b1,
           W2, al2, ar2, b2, P1W, P1b, P2W):
    src = edge_index[0]
    dst = edge_index[1]
    npad_e = EP - E
    pad_i = jnp.arange(npad_e, dtype=jnp.int32)
    srcp = jnp.concatenate([src, (pad_i * 37) % N]).reshape(EP // K, K)
    dstp = jnp.concatenate([dst, N + (pad_i % NPADROWS)]).reshape(EP // K, K)

    al0r, ar0r = al0[None, :], ar0[None, :]
    al1r, ar1r = al1[None, :], ar1[None, :]
    al2r, ar2r = al2[None, :], ar2[None, :]
    b0r, b1r, b2r = b0[None, :], b1[None, :], b2[None, :]
    p1b = P1b[None, :]
    p2w = P2W[:, 0][None, :]
    pad_er = jnp.full((NPADROWS,), -1e30, jnp.float32)

    def edge_phase(feat, er):
        erp = jnp.concatenate([er[:, 0], pad_er])
        num = _sc_edge(feat, erp, srcp, dstp)
        return num[0, :N, :], num[1, :N, :]

    feat, er = _tc_pre0(x, W0, al0r, ar0r)
    n0, n1 = edge_phase(feat, er)
    feat, er, hA = _tc_pre1(n0, n1, b0r, W1, al1r, ar1r)
    n0, n1 = edge_phase(feat, er)
    feat, er, hB = _tc_pre2(n0, n1, hA, b1r, W2, al2r, ar2r)
    n0, n1 = edge_phase(feat, er)
    hC, wsum = _tc_final_a(n0, n1, hB, hA, b2r, P1W, p1b, p2w)
    h_out, beta = _tc_final_b(hA, hB, hC, wsum)
    return (h_out, beta[:, 0:1])


# speedup vs baseline: 42.5817x; 1.0668x over previous
import functools

import jax
import jax.numpy as jnp
from jax import lax
from jax.experimental import pallas as pl
from jax.experimental.pallas import tpu as pltpu
from jax.experimental.pallas import tpu_sc as plsc

N = 10000
E = 320000
IN_FEATS = 128
D = 64
DT = 128
HID = 16

NC = 2
NS = 16
NW = NC * NS
K = 64
CHUNKS = 160
EP = NW * K * CHUNKS
EPT = K * CHUNKS
GRP = 8
NGRP = CHUNKS // GRP
NPADROWS = 112
NPAD = N + NPADROWS
RPT = NPAD // NS
ZCHUNKS = [(i * 64, 64) for i in range(9)] + [(576, 56)]

R = 2000


def _sc_edge_body(feat_hbm, erp_hbm, src2_hbm, dst2_hbm,
                  num_out,
                  er_v, sidxg, didxg, rows_v0, rows_v1, rows_v2, rows_v3,
                  ex_v, shared_num,
                  gsem0, gsem1, gsem2, gsem3, ssem0, ssem1, ssem2, ssem3):
    cid = lax.axis_index("c")
    sid = lax.axis_index("s")
    gwid = cid * NS + sid
    rowbase = gwid * CHUNKS
    bufs = [rows_v0, rows_v1, rows_v2, rows_v3]
    gsems = [gsem0, gsem1, gsem2, gsem3]
    ssems = [ssem0, ssem1, ssem2, ssem3]

    pltpu.sync_copy(erp_hbm, er_v)

    zero16 = jnp.zeros((16,), jnp.float32)

    def zero_body(r, _):
        for j in range(DT // 16):
            rows_v0[r, pl.ds(j * 16, 16)] = zero16
        return 0

    lax.fori_loop(0, K, zero_body, 0)

    for (off, sz) in ZCHUNKS:
        pltpu.sync_copy(rows_v0.at[0:sz],
                        shared_num.at[pl.ds(sid * RPT + off, sz)])
    plsc.subcore_barrier()

    lane = lax.iota(jnp.int32, 16)
    col0 = jnp.zeros((16,), jnp.int32)

    def fetch_group(g):
        p = g % 2
        pltpu.sync_copy(src2_hbm.at[pl.ds(rowbase + g * GRP, GRP)],
                        sidxg.at[p])
        pltpu.sync_copy(dst2_hbm.at[pl.ds(rowbase + g * GRP, GRP)],
                        didxg.at[p])

    def sidx_ref(c):
        return sidxg.at[(c // GRP) % 2, c % GRP]

    def didx_ref(c):
        return didxg.at[(c // GRP) % 2, c % GRP]

    def gather(c, rows_v, gsem):
        pltpu.async_copy(feat_hbm.at[sidx_ref(c)], rows_v, gsem)

    def compute(rows_v, c):
        didxr = didx_ref(c)
        for g in range(K // 16):
            didx = didxr[pl.ds(g * 16, 16)]
            elv = plsc.load_gather(rows_v, [g * 16 + lane, col0 + (D + 1)])
            e = elv + plsc.load_gather(er_v, [didx])
            e = jnp.where(e > 0.0, e, 0.2 * e)
            ex_v[pl.ds(g * 16, 16)] = jnp.exp(e)

        def mul_body(k, _):
            bro = plsc.load_gather(ex_v, [col0 + k])
            for j in range(5):
                rows_v[k, pl.ds(j * 16, 16)] = rows_v[k, pl.ds(j * 16, 16)] * bro
            return 0

        lax.fori_loop(0, K, mul_body, 0, unroll=8)

    def scatter(c, rows_v, ssem):
        pltpu.async_copy(rows_v, shared_num.at[didx_ref(c)], ssem, add=True)

    def swait(c, rows_v, ssem):
        pltpu.make_async_copy(rows_v, shared_num.at[didx_ref(c)], ssem).wait()

    def gwait(c, rows_v, gsem):
        pltpu.make_async_copy(feat_hbm.at[sidx_ref(c)], rows_v, gsem).wait()

    fetch_group(0)
    gather(0, rows_v0, gsem0)
    gather(1, rows_v1, gsem1)

    def quad(i4, _):
        for q in range(4):
            c = i4 * 4 + q
            bq = bufs[q]
            bn = bufs[(q + 2) % 4]
            gwait(c, bq, gsems[q])
            compute(bq, c)
            scatter(c, bq, ssems[q])

            @pl.when(jnp.logical_and(c % GRP == 4, c < (NGRP - 1) * GRP))
            def _():
                fetch_group(c // GRP + 1)

            @pl.when(c + 2 < CHUNKS)
            def _():
                @pl.when(c >= 2)
                def _():
                    swait(c - 2, bn, ssems[(q + 2) % 4])

                gather(c + 2, bn, gsems[(q + 2) % 4])

        return 0

    lax.fori_loop(0, CHUNKS // 4, quad, 0)
    for c in range(CHUNKS - 4, CHUNKS):
        swait(c, bufs[c % 4], ssems[c % 4])
    plsc.subcore_barrier()

    def s2v(i):
        off, sz = ZCHUNKS[i]
        return (shared_num.at[pl.ds(sid * RPT + off, sz)],
                bufs[i % 4].at[0:sz], gsems[i % 4])

    def v2h(i):
        off, sz = ZCHUNKS[i]
        return (bufs[i % 4].at[0:sz],
                num_out.at[cid, pl.ds(sid * RPT + off, sz)], ssems[i % 4])

    nz = len(ZCHUNKS)
    for i in range(nz):
        if i >= 4:
            pltpu.make_async_copy(*v2h(i - 4)).wait()
        pltpu.async_copy(*s2v(i))
        if i >= 1:
            pltpu.make_async_copy(*s2v(i - 1)).wait()
            pltpu.async_copy(*v2h(i - 1))
    pltpu.make_async_copy(*s2v(nz - 1)).wait()
    pltpu.async_copy(*v2h(nz - 1))
    for i in range(nz - 4, nz):
        pltpu.make_async_copy(*v2h(i)).wait()


_sc_edge = functools.partial(
    pl.kernel,
    out_type=jax.ShapeDtypeStruct((NC, NPAD, DT), jnp.float32),
    mesh=plsc.VectorSubcoreMesh(core_axis_name="c", subcore_axis_name="s"),
    compiler_params=pltpu.CompilerParams(needs_layout_passes=False),
    scratch_types=[
        pltpu.VMEM((NPAD,), jnp.float32),
        pltpu.VMEM((2, GRP, K), jnp.int32),
        pltpu.VMEM((2, GRP, K), jnp.int32),
        pltpu.VMEM((K, DT), jnp.float32),
        pltpu.VMEM((K, DT), jnp.float32),
        pltpu.VMEM((K, DT), jnp.float32),
        pltpu.VMEM((K, DT), jnp.float32),
        pltpu.VMEM((K,), jnp.float32),
        pltpu.VMEM_SHARED((NPAD, DT), jnp.float32),
        pltpu.SemaphoreType.DMA,
        pltpu.SemaphoreType.DMA,
        pltpu.SemaphoreType.DMA,
        pltpu.SemaphoreType.DMA,
        pltpu.SemaphoreType.DMA,
        pltpu.SemaphoreType.DMA,
        pltpu.SemaphoreType.DMA,
        pltpu.SemaphoreType.DMA,
    ],
)(_sc_edge_body)


def _l2norm(h):
    n = jnp.sqrt(jnp.sum(h * h, axis=1, keepdims=True))
    return h / jnp.maximum(n, 1e-12)


def _emit(feat, al_ref, ar_ref, feat_ref, er_ref):
    r = feat.shape[0]
    el = jnp.sum(feat * al_ref[...], axis=1, keepdims=True)
    feat_ref[...] = jnp.concatenate(
        [feat, jnp.ones((r, 1), jnp.float32), el,
         jnp.zeros((r, DT - D - 2), jnp.float32)], axis=1)
    er_ref[...] = jnp.sum(feat * ar_ref[...], axis=1, keepdims=True)


def _pre0_body(x_ref, w_ref, al_ref, ar_ref, feat_ref, er_ref):
    hn = _l2norm(x_ref[...])
    feat = jnp.dot(hn, w_ref[...], preferred_element_type=jnp.float32)
    _emit(feat, al_ref, ar_ref, feat_ref, er_ref)


def _tc_pre0(x, w, al, ar):
    return pl.pallas_call(
        _pre0_body,
        grid=(N // R,),
        in_specs=[
            pl.BlockSpec((R, IN_FEATS), lambda i: (i, 0)),
            pl.BlockSpec((IN_FEATS, D), lambda i: (0, 0)),
            pl.BlockSpec((1, D), lambda i: (0, 0)),
            pl.BlockSpec((1, D), lambda i: (0, 0)),
        ],
        out_specs=[
            pl.BlockSpec((R, DT), lambda i: (i, 0)),
            pl.BlockSpec((R, 1), lambda i: (i, 0)),
        ],
        out_shape=[
            jax.ShapeDtypeStruct((N, DT), jnp.float32),
            jax.ShapeDtypeStruct((N, 1), jnp.float32),
        ],
    )(x, w, al, ar)


def _combine(num_ref, b_ref):
    n0 = num_ref[0]
    n1 = num_ref[1]
    s = jnp.maximum(n0[:, D:D + 1] + n1[:, D:D + 1], 1e-12)
    return (n0[:, 0:D] + n1[:, 0:D]) / s + b_ref[...]


def _make_pre_mid(with_prev):
    def body(*refs):
        if with_prev:
            (num_ref, hprev_ref, b_ref,
             w_ref, al_ref, ar_ref, feat_ref, er_ref, hacc_ref) = refs
        else:
            (num_ref, b_ref,
             w_ref, al_ref, ar_ref, feat_ref, er_ref, hacc_ref) = refs
        rst = _combine(num_ref, b_ref)
        hacc = rst + hprev_ref[...] if with_prev else rst
        hacc_ref[...] = hacc
        hn = _l2norm(hacc)
        feat = jnp.dot(hn, w_ref[...], preferred_element_type=jnp.float32)
        _emit(feat, al_ref, ar_ref, feat_ref, er_ref)

    rn = pl.BlockSpec((NC, R, DT), lambda i: (0, i, 0))
    rt = pl.BlockSpec((R, DT), lambda i: (i, 0))
    rd = pl.BlockSpec((R, D), lambda i: (i, 0))
    r1 = pl.BlockSpec((R, 1), lambda i: (i, 0))
    full1d = pl.BlockSpec((1, D), lambda i: (0, 0))
    in_specs = [rn] + ([rd] if with_prev else []) + [
        full1d, pl.BlockSpec((D, D), lambda i: (0, 0)), full1d, full1d]

    def run(*args):
        return pl.pallas_call(
            body,
            grid=(N // R,),
            in_specs=in_specs,
            out_specs=[rt, r1, rd],
            out_shape=[
                jax.ShapeDtypeStruct((N, DT), jnp.float32),
                jax.ShapeDtypeStruct((N, 1), jnp.float32),
                jax.ShapeDtypeStruct((N, D), jnp.float32),
            ],
        )(*args)

    return run


_tc_pre1 = _make_pre_mid(False)
_tc_pre2 = _make_pre_mid(True)


def _score(h, p1w_ref, p1b_ref, p2w_ref):
    t = jnp.tanh(jnp.dot(h, p1w_ref[...], preferred_element_type=jnp.float32)
                 + p1b_ref[...])
    return jnp.sum(t * p2w_ref[...])


def _final_a_body(num_ref, hB_ref, hA_ref, b_ref,
                  p1w_ref, p1b_ref, p2w_ref, hC_ref, wsum_ref):
    i = pl.program_id(0)
    rst = _combine(num_ref, b_ref)
    hC = hB_ref[...] + rst
    hC_ref[...] = hC
    w0 = _score(hA_ref[...], p1w_ref, p1b_ref, p2w_ref)
    w1 = _score(hB_ref[...], p1w_ref, p1b_ref, p2w_ref)
    w2 = _score(hC, p1w_ref, p1b_ref, p2w_ref)
    part = jnp.broadcast_to(jnp.stack([w0, w1, w2])[:, None], (3, 128))

    @pl.when(i == 0)
    def _():
        wsum_ref[...] = jnp.zeros_like(wsum_ref)

    wsum_ref[...] += part


def _tc_final_a(num, hB, hA, b, p1w, p1b, p2w):
    rn = pl.BlockSpec((NC, R, DT), lambda i: (0, i, 0))
    rd = pl.BlockSpec((R, D), lambda i: (i, 0))
    return pl.pallas_call(
        _final_a_body,
        grid=(N // R,),
        in_specs=[rn, rd, rd,
                  pl.BlockSpec((1, D), lambda i: (0, 0)),
                  pl.BlockSpec((D, HID), lambda i: (0, 0)),
                  pl.BlockSpec((1, HID), lambda i: (0, 0)),
                  pl.BlockSpec((1, HID), lambda i: (0, 0))],
        out_specs=[rd, pl.BlockSpec((3, 128), lambda i: (0, 0))],
        out_shape=[
            jax.ShapeDtypeStruct((N, D), jnp.float32),
            jax.ShapeDtypeStruct((3, 128), jnp.float32),
        ],
    )(num, hB, hA, b, p1w, p1b, p2w)


def _final_b_body(hA_ref, hB_ref, hC_ref, wsum_ref, hout_ref, beta_ref):
    w = wsum_ref[...] * (1.0 / N)
    m = jnp.max(w[:, 0:1])
    ew = jnp.exp(w - m)
    beta = ew / jnp.sum(ew[:, 0:1])
    beta_ref[...] = beta
    bc = beta[:, 0:D]
    hsum = (hA_ref[...] * bc[0:1] + hB_ref[...] * bc[1:2]
            + hC_ref[...] * bc[2:3])
    hout_ref[...] = _l2norm(hsum)


def _tc_final_b(hA, hB, hC, wsum):
    rd = pl.BlockSpec((R, D), lambda i: (i, 0))
    return pl.pallas_call(
        _final_b_body,
        grid=(N // R,),
        in_specs=[rd, rd, rd, pl.BlockSpec((3, 128), lambda i: (0, 0))],
        out_specs=[rd, pl.BlockSpec((3, 128), lambda i: (0, 0))],
        out_shape=[
            jax.ShapeDtypeStruct((N, D), jnp.float32),
            jax.ShapeDtypeStruct((3, 128), jnp.float32),
        ],
    )(hA, hB, hC, wsum)


def kernel(x, edge_index, W0, al0, ar0, b0, W1, al1, ar1, b1,
           W2, al2, ar2, b2, P1W, P1b, P2W):
    src = edge_index[0]
    dst = edge_index[1]
    npad_e = EP - E
    pad_i = jnp.arange(npad_e, dtype=jnp.int32)
    srcp = jnp.concatenate([src, (pad_i * 37) % N]).reshape(EP // K, K)
    dstp = jnp.concatenate([dst, N + (pad_i % NPADROWS)]).reshape(EP // K, K)

    al0r, ar0r = al0[None, :], ar0[None, :]
    al1r, ar1r = al1[None, :], ar1[None, :]
    al2r, ar2r = al2[None, :], ar2[None, :]
    b0r, b1r, b2r = b0[None, :], b1[None, :], b2[None, :]
    p1b = P1b[None, :]
    p2w = P2W[:, 0][None, :]
    pad_er = jnp.full((NPADROWS,), -1e30, jnp.float32)

    def edge_phase(feat, er):
        erp = jnp.concatenate([er[:, 0], pad_er])
        return _sc_edge(feat, erp, srcp, dstp)

    feat, er = _tc_pre0(x, W0, al0r, ar0r)
    num = edge_phase(feat, er)
    feat, er, hA = _tc_pre1(num, b0r, W1, al1r, ar1r)
    num = edge_phase(feat, er)
    feat, er, hB = _tc_pre2(num, hA, b1r, W2, al2r, ar2r)
    num = edge_phase(feat, er)
    hC, wsum = _tc_final_a(num, hB, hA, b2r, P1W, p1b, p2w)
    h_out, beta = _tc_final_b(hA, hB, hC, wsum)
    return (h_out, beta[:, 0:1])
